# baseline (speedup 1.0000x reference)
import jax
import jax.numpy as jnp
from jax import lax
from jax.experimental import pallas as pl
from jax.experimental.pallas import tpu as pltpu
from jax.experimental.pallas import tpu_sc as plsc

NUM_EMB = 100000
DIM = 128
NC = 2
NS = 16
NW = NC * NS
CH = 128
NBUF = 4
TOTAL = 4096 * 200
PER_W = TOTAL // NW
NCHUNK = PER_W // CH


def _body(idx_hbm, table_hbm, out_hbm, idx_v, rows, gsems, osems):
    wid = lax.axis_index("s") * NC + lax.axis_index("c")
    pltpu.sync_copy(idx_hbm.at[wid], idx_v)
    base = wid * PER_W

    def gather(ch, b):
        pltpu.async_copy(table_hbm.at[idx_v.at[ch]], rows[b], gsems[b])

    def gather_wait(ch, b):
        pltpu.make_async_copy(table_hbm.at[idx_v.at[ch]], rows[b], gsems[b]).wait()

    def write(ch, b):
        pltpu.async_copy(rows[b], out_hbm.at[pl.ds(base + ch * CH, CH)], osems[b])

    def write_wait(ch, b):
        pltpu.make_async_copy(
            rows[b], out_hbm.at[pl.ds(base + ch * CH, CH)], osems[b]).wait()

    gather(0, 0)
    gather(1, 1)
    for ch in (0, 1):
        gather_wait(ch, ch)
        write(ch, ch)
        gather(ch + 2, ch + 2)

    def round_(r, carry):
        for j in range(NBUF):
            ch = 4 * r + 2 + j
            b = (2 + j) % NBUF
            gather_wait(ch, b)
            write(ch, b)
            nxt = ch + 2
            nb = (b + 2) % NBUF
            write_wait(nxt - NBUF, nb)
            gather(nxt, nb)
        return carry

    lax.fori_loop(0, (NCHUNK - 4) // NBUF, round_, 0)

    for ch in (NCHUNK - 2, NCHUNK - 1):
        b = ch % NBUF
        gather_wait(ch, b)
        write(ch, b)
    for ch in range(NCHUNK - NBUF, NCHUNK):
        write_wait(ch, ch % NBUF)


_sc_gather = pl.kernel(
    _body,
    out_type=jax.ShapeDtypeStruct((TOTAL, DIM), jnp.float32),
    mesh=plsc.VectorSubcoreMesh(core_axis_name="c", subcore_axis_name="s"),
    scratch_types=[
        pltpu.VMEM((NCHUNK, CH), jnp.int32),
        [pltpu.VMEM((CH, DIM), jnp.float32) for _ in range(NBUF)],
        [pltpu.SemaphoreType.DMA for _ in range(NBUF)],
        [pltpu.SemaphoreType.DMA for _ in range(NBUF)],
    ],
)


def kernel(input, embed_weight):
    idx = input.reshape(NW, NCHUNK, CH).astype(jnp.int32)
    out = _sc_gather(idx, embed_weight)
    return out.reshape(input.shape[0], input.shape[1], DIM)
